# Initial kernel scaffold; baseline (speedup 1.0000x reference)
import jax
import jax.numpy as jnp
from jax.experimental import pallas as pl


def kernel(target, context, negative_samples, target_table, context_table):
    raise NotImplementedError("write your pallas kernel here")


# speedup vs baseline: 4.7460x; 4.7460x over previous
import functools

import jax
import jax.numpy as jnp
from jax import lax
from jax.experimental import pallas as pl
from jax.experimental.pallas import tpu as pltpu
from jax.experimental.pallas import tpu_sc as plsc

B = 16384
D = 64
NNEG = 20
NC = 2
NS = 16
L = 16
NW = NC * NS
BPW = B // NW
CHUNK = 64
NCHUNK = BPW // CHUNK
NIDX_BLK = 128
NBLK = CHUNK * NNEG // NIDX_BLK
DK = D // L


def _sg_body(t_idx_hbm, c_idx_hbm, n_idx_hbm, t_tab, c_tab,
             pos_hbm, neg_hbm,
             t_idx_v, c_idx_v, n_idx_v, t_rows, c_rows, n_rows,
             pos_v, neg_v, sem):
    wid = lax.axis_index("s") * NC + lax.axis_index("c")
    lane = lax.iota(jnp.int32, L)
    last = lane == (L - 1)

    def chunk_body(ch, carry):
        base = wid * BPW + ch * CHUNK
        pltpu.sync_copy(t_idx_hbm.at[pl.ds(base, CHUNK)], t_idx_v)
        pltpu.sync_copy(c_idx_hbm.at[pl.ds(base, CHUNK)], c_idx_v)
        pltpu.sync_copy(n_idx_hbm.at[pl.ds(base * NNEG, CHUNK * NNEG)],
                        n_idx_v)
        dmas = [pltpu.async_copy(t_tab.at[t_idx_v], t_rows, sem),
                pltpu.async_copy(c_tab.at[c_idx_v], c_rows, sem)]
        for k in range(NBLK):
            dmas.append(pltpu.async_copy(
                c_tab.at[n_idx_v.at[pl.ds(k * NIDX_BLK, NIDX_BLK)]],
                n_rows.at[pl.ds(k * NIDX_BLK, NIDX_BLK)], sem))
        for dma in dmas:
            dma.wait()

        def elem_body(e, ecarry):
            tv = [t_rows[e, pl.ds(k * L, L)] for k in range(DK)]
            prod = tv[0] * c_rows[e, pl.ds(0, L)]
            for k in range(1, DK):
                prod = prod + tv[k] * c_rows[e, pl.ds(k * L, L)]
            plsc.store_scatter(pos_v, [jnp.full((L,), e, jnp.int32)],
                               plsc.cumsum(prod), mask=last)
            for j in range(NNEG):
                r = e * NNEG + j
                prod = tv[0] * n_rows[r, pl.ds(0, L)]
                for k in range(1, DK):
                    prod = prod + tv[k] * n_rows[r, pl.ds(k * L, L)]
                plsc.store_scatter(neg_v, [jnp.full((L,), r, jnp.int32)],
                                   plsc.cumsum(prod), mask=last)
            return ecarry

        lax.fori_loop(0, CHUNK, elem_body, 0)
        pltpu.sync_copy(pos_v, pos_hbm.at[pl.ds(base, CHUNK)])
        pltpu.sync_copy(neg_v, neg_hbm.at[pl.ds(base * NNEG, CHUNK * NNEG)])
        return carry

    lax.fori_loop(0, NCHUNK, chunk_body, 0)


_sg_kernel = functools.partial(
    pl.kernel,
    mesh=plsc.VectorSubcoreMesh(core_axis_name="c", subcore_axis_name="s"),
    out_type=[jax.ShapeDtypeStruct((B,), jnp.float32),
              jax.ShapeDtypeStruct((B * NNEG,), jnp.float32)],
    scratch_types=[
        pltpu.VMEM((CHUNK,), jnp.int32),
        pltpu.VMEM((CHUNK,), jnp.int32),
        pltpu.VMEM((CHUNK * NNEG,), jnp.int32),
        pltpu.VMEM((CHUNK, D), jnp.float32),
        pltpu.VMEM((CHUNK, D), jnp.float32),
        pltpu.VMEM((CHUNK * NNEG, D), jnp.float32),
        pltpu.VMEM((CHUNK,), jnp.float32),
        pltpu.VMEM((CHUNK * NNEG,), jnp.float32),
        pltpu.SemaphoreType.DMA,
    ],
    compiler_params=pltpu.CompilerParams(needs_layout_passes=False,
                                         use_tc_tiling_on_sc=False),
)(_sg_body)


def kernel(target, context, negative_samples, target_table, context_table):
    pos, neg = _sg_kernel(target.astype(jnp.int32),
                          context.astype(jnp.int32),
                          negative_samples.reshape(-1).astype(jnp.int32),
                          target_table, context_table)
    return pos, neg.reshape(B, NNEG)
